# baseline (speedup 1.0000x reference)
import functools

import jax
import jax.numpy as jnp
from jax import lax
from jax.experimental import pallas as pl
from jax.experimental.pallas import tpu as pltpu
from jax.experimental.pallas import tpu_sc as plsc

NC = 2
NS = 16
NW = NC * NS
D = 64
L = 16
T = 200
BB = 128


@functools.cache
def _make_kernel():
    mesh = plsc.VectorSubcoreMesh(core_axis_name="c", subcore_axis_name="s")

    @functools.partial(
        pl.kernel,
        out_type=jax.ShapeDtypeStruct((T, D // 8, NW, 8 * BB), jnp.float32),
        mesh=mesh,
        scratch_types=[
            pltpu.VMEM((T, BB), jnp.int32),
            pltpu.VMEM((4, BB, D), jnp.float32),
            pltpu.VMEM((4, 2, D * BB), jnp.float32),
            pltpu.SemaphoreType.DMA,
            pltpu.SemaphoreType.DMA,
        ],
        compiler_params=pltpu.CompilerParams(use_tc_tiling_on_sc=False,
                                             needs_layout_passes=False),
    )
    def gather_kernel(idx_hbm, table_hbm, out_hbm,
                      idx_v, rows_v, til_v, gsem, osem):
        wid = lax.axis_index("s") * NC + lax.axis_index("c")
        pltpu.sync_copy(idx_hbm.at[:, pl.ds(wid * BB, BB)], idx_v)

        iota = lax.iota(jnp.int32, L)

        def fire_g(t, rb):
            pltpu.async_copy(table_hbm.at[idx_v.at[t]], rows_v.at[rb], gsem)

        def drain_g(rb):
            pltpu.make_async_copy(
                table_hbm.at[idx_v.at[0]], rows_v.at[rb], gsem).wait()

        def transpose(rb, pb, s):
            rows = rows_v.at[rb]
            til = til_v.at[pb, s]

            @plsc.parallel_loop(0, L, step=1, unroll=2)
            def _(k):
                rot = (iota + k) & (L - 1)
                st_base = (rot << 7) + iota
                for c0 in range(D // L):
                    cvec = rot + (c0 * L)
                    for j in range(BB // L):
                        v = plsc.load_gather(rows, [iota + j * L, cvec])
                        plsc.store_scatter(
                            til, [st_base + (c0 * L * BB + j * L)], v)

        def fire_o(t0, pb):
            for cg in range(D // 8):
                pltpu.async_copy(
                    til_v.at[pb, :, pl.ds(cg * 8 * BB, 8 * BB)],
                    out_hbm.at[pl.ds(t0, 2), cg, wid], osem)

        def wait_o():
            for cg in range(D // 8):
                pltpu.make_async_copy(
                    til_v.at[0, :, pl.ds(0, 8 * BB)],
                    out_hbm.at[pl.ds(0, 2), 0, wid], osem).wait()

        fire_g(0, 0)
        fire_g(1, 1)
        fire_g(2, 2)

        def step(t, j):
            rb = j % 4
            pb = j // 2
            drain_g(rb)

            @pl.when(t < T - 3)
            def _():
                fire_g(t + 3, (rb + 3) % 4)

            if j % 2 == 0:
                @pl.when(t >= 8)
                def _():
                    wait_o()

            transpose(rb, pb, j % 2)
            if j % 2 == 1:
                fire_o(t - 1, pb)

        def body(i, carry):
            t = 8 * i
            for j in range(8):
                step(t + j, j)
            return carry

        lax.fori_loop(0, T // 8, body, 0)
        for _ in range(4):
            wait_o()

    return gather_kernel


@jax.jit
def kernel(x, pe):
    idx = x.reshape(x.shape[0], x.shape[1]).astype(jnp.int32).T
    a = _make_kernel()(idx, pe)
    a = a.reshape(T, D // 8, NW, 8, BB).transpose(2, 4, 0, 1, 3)
    return a.reshape(x.shape[0], x.shape[1], pe.shape[1])


# speedup vs baseline: 1.1400x; 1.1400x over previous
import functools

import jax
import jax.numpy as jnp
from jax import lax
from jax.experimental import pallas as pl
from jax.experimental.pallas import tpu as pltpu
from jax.experimental.pallas import tpu_sc as plsc

NC = 2
NS = 16
NW = NC * NS
D = 64
L = 16
T = 200
BB = 128


@functools.cache
def _make_kernel():
    mesh = plsc.VectorSubcoreMesh(core_axis_name="c", subcore_axis_name="s")

    @functools.partial(
        pl.kernel,
        out_type=jax.ShapeDtypeStruct((T, D // 8, NW, 8 * BB), jnp.float32),
        mesh=mesh,
        scratch_types=[
            pltpu.VMEM((T, BB), jnp.int32),
            pltpu.VMEM((4, BB, D), jnp.float32),
            pltpu.VMEM((4, D * BB), jnp.float32),
            pltpu.SemaphoreType.DMA,
            pltpu.SemaphoreType.DMA,
        ],
        compiler_params=pltpu.CompilerParams(use_tc_tiling_on_sc=False,
                                             needs_layout_passes=False),
    )
    def gather_kernel(idx_hbm, table_hbm, out_hbm,
                      idx_v, rows_v, til_v, gsem, osem):
        wid = lax.axis_index("s") * NC + lax.axis_index("c")

        iota = lax.iota(jnp.int32, L)

        def fire_g(t, rb):
            pltpu.async_copy(table_hbm.at[idx_v.at[t]], rows_v.at[rb], gsem)

        def drain_g(rb):
            pltpu.make_async_copy(
                table_hbm.at[idx_v.at[0]], rows_v.at[rb], gsem).wait()

        def transpose(rb, tb):
            rows = rows_v.at[rb]
            til = til_v.at[tb]

            @plsc.parallel_loop(0, L, step=1, unroll=2)
            def _(k):
                rot = (iota + k) & (L - 1)
                st_base = (rot << 7) + iota
                for c0 in range(D // L):
                    cvec = rot + (c0 * L)
                    for j in range(BB // L):
                        v = plsc.load_gather(rows, [iota + j * L, cvec])
                        plsc.store_scatter(
                            til, [st_base + (c0 * L * BB + j * L)], v)

        def fire_o(t, tb):
            for cg in range(D // 8):
                pltpu.async_copy(til_v.at[tb, pl.ds(cg * 8 * BB, 8 * BB)],
                                 out_hbm.at[t, cg, wid], osem)

        def wait_o():
            for cg in range(D // 8):
                pltpu.make_async_copy(
                    til_v.at[0, pl.ds(0, 8 * BB)],
                    out_hbm.at[0, 0, wid], osem).wait()

        pltpu.sync_copy(idx_hbm.at[pl.ds(0, 8), pl.ds(wid * BB, BB)],
                        idx_v.at[pl.ds(0, 8)])
        fire_g(0, 0)
        fire_g(1, 1)
        fire_g(2, 2)
        pltpu.sync_copy(idx_hbm.at[pl.ds(8, T - 8), pl.ds(wid * BB, BB)],
                        idx_v.at[pl.ds(8, T - 8)])

        def step(t, buf):
            drain_g(buf)

            @pl.when(t < T - 3)
            def _():
                fire_g(t + 3, (buf + 3) % 4)

            @pl.when(t >= 4)
            def _():
                wait_o()

            transpose(buf, buf)
            fire_o(t, buf)

        def body(i, carry):
            t = 4 * i
            for j in range(4):
                step(t + j, j)
            return carry

        lax.fori_loop(0, T // 4, body, 0)
        for _ in range(4):
            wait_o()

    return gather_kernel


@jax.jit
def kernel(x, pe):
    idx = x.reshape(x.shape[0], x.shape[1]).astype(jnp.int32).T
    a = _make_kernel()(idx, pe)
    a = a.reshape(T, D // 8, NW, 8, BB).transpose(2, 4, 0, 1, 3)
    return a.reshape(x.shape[0], x.shape[1], pe.shape[1])
